# baseline (speedup 1.0000x reference)
import jax
import jax.numpy as jnp
from jax import lax
from jax.experimental import pallas as pl
from jax.experimental.pallas import tpu as pltpu
from jax.experimental.pallas import tpu_sc as plsc

N = 32768
E = 524288
D_IN = 4
D_H = 128
NB = 128
FIS = N // NB * D_H
NC = 2
NS = 16
FC = 16
NCHUNK = D_H // FC
ROWS = E // 128
NBLK = N // 16
HIGH = lax.Precision.HIGHEST

def _sc_mesh():
    return plsc.VectorSubcoreMesh(
        core_axis_name="c", subcore_axis_name="s",
        num_cores=NC, num_subcores=NS)


def _deg_body(dst_hbm, ones_hbm, zero1_hbm, out_hbm, idx_v, ones_v, acc_sh):
    c = lax.axis_index("c")
    s = lax.axis_index("s")
    pltpu.sync_copy(zero1_hbm, acc_sh.at[pl.ds(s * NBLK, NBLK)])
    pltpu.sync_copy(ones_hbm, ones_v)
    row0 = c * (ROWS // NC) + s * 128
    pltpu.sync_copy(dst_hbm.at[pl.ds(row0, 128)], idx_v)
    plsc.subcore_barrier()

    def body(j, carry):
        pltpu.sync_copy(ones_v, acc_sh.at[idx_v.at[j]], add=True)
        return carry

    lax.fori_loop(0, 128, body, 0)
    plsc.subcore_barrier()
    pltpu.sync_copy(acc_sh.at[pl.ds(s * NBLK, NBLK)],
                    out_hbm.at[c, pl.ds(s * NBLK, NBLK)])


def _sc_degree(dst2d, ones, zero1):
    return pl.kernel(
        _deg_body,
        out_type=jax.ShapeDtypeStruct((NC, N), jnp.float32),
        mesh=_sc_mesh(),
        scratch_types=[
            pltpu.VMEM((128, 128), jnp.int32),
            pltpu.VMEM((128,), jnp.float32),
            pltpu.VMEM_SHARED((N,), jnp.float32),
        ],
    )(dst2d, ones, zero1)


_EB = 1024
EROWS = E // _EB
_NBUF = 3
_LOOK = 2


def _prop_body(g_hbm, srcall_hbm, dst_hbm, zero_hbm, out_hbm,
               src_v, dst_v, stage, acc_sh, gsem, ssem):
    c = lax.axis_index("c")
    s = lax.axis_index("s")
    nbh = 16
    nsc = _EB // 128
    for k in range(NCHUNK // NC):
        chunk = c * (NCHUNK // NC) + k
        pltpu.sync_copy(srcall_hbm.at[chunk, pl.ds(s * 2 * nbh, 2 * nbh)],
                        src_v)
        for z in range(NBLK // 512):
            pltpu.sync_copy(zero_hbm, acc_sh.at[pl.ds(s * NBLK + z * 512, 512)])
        plsc.subcore_barrier()

        for half in range(2):
            pltpu.sync_copy(
                dst_hbm.at[pl.ds(s * 256 + half * 128, 128)], dst_v)

            def issue_gather(j, half=half):
                pltpu.async_copy(
                    g_hbm.at[src_v.at[pl.ds(half * nbh + j, 1)]],
                    stage.at[lax.rem(j, _NBUF)], gsem)

            def wait_gather(j, half=half):
                pltpu.make_async_copy(
                    g_hbm.at[src_v.at[pl.ds(half * nbh + j, 1)]],
                    stage.at[lax.rem(j, _NBUF)], gsem).wait()

            def issue_scatters(j):
                slot = lax.rem(j, _NBUF)
                for i in range(nsc):
                    pltpu.async_copy(
                        stage.at[slot, 0, pl.ds(i * 128, 128), 0],
                        acc_sh.at[dst_v.at[j * nsc + i]], ssem, add=True)

            def wait_scatters(j):
                slot = lax.rem(j, _NBUF)
                for i in range(nsc):
                    pltpu.make_async_copy(
                        stage.at[slot, 0, pl.ds(i * 128, 128), 0],
                        acc_sh.at[dst_v.at[j * nsc + i]], ssem).wait()

            for j in range(_LOOK):
                issue_gather(j)

            def body(j, carry):
                @pl.when(j >= _NBUF - _LOOK)
                def _free():
                    wait_scatters(j - (_NBUF - _LOOK))

                @pl.when(j + _LOOK < nbh)
                def _ahead():
                    issue_gather(j + _LOOK)

                wait_gather(j)
                issue_scatters(j)
                return carry

            lax.fori_loop(0, nbh, body, 0)
            for j in range(_NBUF - _LOOK):
                wait_scatters(nbh - (_NBUF - _LOOK) + j)
        plsc.subcore_barrier()
        pltpu.sync_copy(acc_sh.at[pl.ds(s * NBLK, NBLK)],
                        out_hbm.at[chunk, pl.ds(s * NBLK, NBLK)])
        if k + 1 < NCHUNK // NC:
            plsc.subcore_barrier()


def _sc_prop(g4, src_all, dst2d, zero2):
    return pl.kernel(
        _prop_body,
        out_type=jax.ShapeDtypeStruct((NCHUNK, N, FC), jnp.float32),
        mesh=_sc_mesh(),
        compiler_params=pltpu.CompilerParams(use_tc_tiling_on_sc=False),
        scratch_types=[
            pltpu.VMEM((EROWS // NS, _EB), jnp.int32),
            pltpu.VMEM((128, 128), jnp.int32),
            pltpu.VMEM((_NBUF, 1, _EB, 1, FC), jnp.float32),
            pltpu.VMEM_SHARED((N, FC), jnp.float32),
            pltpu.SemaphoreType.DMA,
            pltpu.SemaphoreType.DMA,
        ],
    )(g4, src_all, dst2d, zero2)


def _tc_idx(src2d):
    def body(src_ref, out_ref):
        k = pl.program_id(0)
        out_ref[...] = src_ref[...][None] + k * N

    blk = EROWS // 8
    return pl.pallas_call(
        body,
        grid=(NCHUNK, 8),
        in_specs=[pl.BlockSpec((blk, _EB), lambda k, i: (i, 0))],
        out_specs=pl.BlockSpec((1, blk, _EB), lambda k, i: (k, i, 0)),
        out_shape=jax.ShapeDtypeStruct((NCHUNK, EROWS, _EB), jnp.int32),
    )(src2d)


def _tc_pre(degT, x, W1):
    def body(deg_ref, x_ref, w1_ref, g1_ref, dinv_ref):
        deg = jnp.sum(deg_ref[...], axis=1, keepdims=True) + 1.0
        dinv = lax.rsqrt(deg)
        h = jnp.dot(x_ref[...], w1_ref[...],
                    preferred_element_type=jnp.float32, precision=HIGH)
        g1_ref[...] = h * dinv
        dinv_ref[...] = dinv

    return pl.pallas_call(
        body,
        grid=(16,),
        in_specs=[
            pl.BlockSpec((NBLK, NC), lambda i: (i, 0)),
            pl.BlockSpec((NBLK, D_IN), lambda i: (i, 0)),
            pl.BlockSpec((D_IN, D_H), lambda i: (0, 0)),
        ],
        out_specs=[
            pl.BlockSpec((NBLK, D_H), lambda i: (i, 0)),
            pl.BlockSpec((NBLK, 1), lambda i: (i, 0)),
        ],
        out_shape=[
            jax.ShapeDtypeStruct((N, D_H), jnp.float32),
            jax.ShapeDtypeStruct((N, 1), jnp.float32),
        ],
    )(degT, x, W1)


def _tc_mid(S1, g1, dinv, b1, W2):
    def body(s_ref, g_ref, dinv_ref, b1_ref, w2_ref, g2_ref):
        h1 = jnp.maximum(
            dinv_ref[...] * (s_ref[...] + g_ref[...]) + b1_ref[...], 0.0)
        g2_ref[...] = dinv_ref[...] * jnp.dot(
            h1, w2_ref[...], preferred_element_type=jnp.float32, precision=HIGH)

    return pl.pallas_call(
        body,
        grid=(16,),
        in_specs=[
            pl.BlockSpec((NBLK, D_H), lambda i: (i, 0)),
            pl.BlockSpec((NBLK, D_H), lambda i: (i, 0)),
            pl.BlockSpec((NBLK, 1), lambda i: (i, 0)),
            pl.BlockSpec((1, D_H), lambda i: (0, 0)),
            pl.BlockSpec((D_H, D_H), lambda i: (0, 0)),
        ],
        out_specs=pl.BlockSpec((NBLK, D_H), lambda i: (i, 0)),
        out_shape=jax.ShapeDtypeStruct((N, D_H), jnp.float32),
    )(S1, g1, dinv, b1, W2)


def _tc_h2(S2, g2, dinv, b2):
    def body(s_ref, g_ref, dinv_ref, b2_ref, h2_ref):
        h2_ref[...] = jnp.maximum(
            dinv_ref[...] * (s_ref[...] + g_ref[...]) + b2_ref[...], 0.0)

    return pl.pallas_call(
        body,
        grid=(16,),
        in_specs=[
            pl.BlockSpec((NBLK, D_H), lambda i: (i, 0)),
            pl.BlockSpec((NBLK, D_H), lambda i: (i, 0)),
            pl.BlockSpec((NBLK, 1), lambda i: (i, 0)),
            pl.BlockSpec((1, D_H), lambda i: (0, 0)),
        ],
        out_specs=pl.BlockSpec((NBLK, D_H), lambda i: (i, 0)),
        out_shape=jax.ShapeDtypeStruct((N, D_H), jnp.float32),
    )(S2, g2, dinv, b2)


def _tc_heads(flat, Wab, ba1, bv1, Wa2, ba2, Wv2, bv2):
    nsteps = FIS // NBLK

    def body(flat_ref, wab_ref, ba1_ref, bv1_ref, wa2_ref, ba2_ref,
             wv2_ref, bv2_ref, act_ref, val_ref, acc_ref):
        i = pl.program_id(0)

        @pl.when(i == 0)
        def _init():
            acc_ref[...] = jnp.zeros_like(acc_ref)

        acc_ref[...] += jnp.dot(flat_ref[...], wab_ref[...],
                                preferred_element_type=jnp.float32,
                                precision=HIGH)

        @pl.when(i == nsteps - 1)
        def _fin():
            acc = acc_ref[...]
            a = jnp.maximum(acc[:, :64] + ba1_ref[...], 0.0)
            logits = jnp.dot(a, wa2_ref[...],
                             preferred_element_type=jnp.float32,
                             precision=HIGH) + ba2_ref[...]
            m = jnp.max(logits, axis=1, keepdims=True)
            lse = m + jnp.log(jnp.sum(jnp.exp(logits - m), axis=1,
                                      keepdims=True))
            act_ref[...] = logits - lse
            v = jnp.maximum(acc[:, 64:] + bv1_ref[...], 0.0)
            val_ref[...] = jnp.tanh(
                jnp.dot(v, wv2_ref[...], preferred_element_type=jnp.float32,
                        precision=HIGH) + bv2_ref[...])

    return pl.pallas_call(
        body,
        grid=(nsteps,),
        in_specs=[
            pl.BlockSpec((NB, NBLK), lambda i: (0, i)),
            pl.BlockSpec((NBLK, 2 * 64), lambda i: (i, 0)),
            pl.BlockSpec((1, 64), lambda i: (0, 0)),
            pl.BlockSpec((1, 64), lambda i: (0, 0)),
            pl.BlockSpec((64, 256), lambda i: (0, 0)),
            pl.BlockSpec((1, 256), lambda i: (0, 0)),
            pl.BlockSpec((64, 1), lambda i: (0, 0)),
            pl.BlockSpec((1, 1), lambda i: (0, 0)),
        ],
        out_specs=[
            pl.BlockSpec((NB, 256), lambda i: (0, 0)),
            pl.BlockSpec((NB, 1), lambda i: (0, 0)),
        ],
        out_shape=[
            jax.ShapeDtypeStruct((NB, 256), jnp.float32),
            jax.ShapeDtypeStruct((NB, 1), jnp.float32),
        ],
        scratch_shapes=[pltpu.VMEM((NB, 2 * 64), jnp.float32)],
    )(flat, Wab, ba1, bv1, Wa2, ba2, Wv2, bv2)


def kernel(x, edge_index, W1, b1, W2, b2, Wa1, ba1, Wa2, ba2, Wv1, bv1,
           Wv2, bv2):
    src2d = edge_index[0].reshape(ROWS, 128)
    dst2d = edge_index[1].reshape(ROWS, 128)
    ones = jnp.ones((128,), jnp.float32)
    zero1 = jnp.zeros((NBLK,), jnp.float32)
    zero2 = jnp.zeros((512, FC), jnp.float32)

    deg2 = _sc_degree(dst2d, ones, zero1)
    src_all = _tc_idx(src2d.reshape(EROWS, _EB))
    g1, dinv = _tc_pre(deg2.T, x, W1)

    g1c = jnp.moveaxis(g1.reshape(N, NCHUNK, FC), 1, 0)
    S1c = _sc_prop(g1c.reshape(1, NCHUNK * N, 1, FC), src_all, dst2d, zero2)
    S1 = jnp.moveaxis(S1c, 0, 1).reshape(N, D_H)

    g2 = _tc_mid(S1, g1, dinv, b1.reshape(1, D_H), W2)
    g2c = jnp.moveaxis(g2.reshape(N, NCHUNK, FC), 1, 0)
    S2c = _sc_prop(g2c.reshape(1, NCHUNK * N, 1, FC), src_all, dst2d, zero2)
    S2 = jnp.moveaxis(S2c, 0, 1).reshape(N, D_H)

    h2 = _tc_h2(S2, g2, dinv, b2.reshape(1, D_H))
    flat = h2.reshape(NB, FIS)
    Wab = jnp.concatenate([Wa1, Wv1], axis=1)
    x_act, x_val = _tc_heads(flat, Wab, ba1.reshape(1, 64),
                             bv1.reshape(1, 64), Wa2, ba2.reshape(1, 256),
                             Wv2, bv2.reshape(1, 1))
    return (x_act, x_val)


# speedup vs baseline: 1.7396x; 1.7396x over previous
import jax
import jax.numpy as jnp
from jax import lax
from jax.experimental import pallas as pl
from jax.experimental.pallas import tpu as pltpu
from jax.experimental.pallas import tpu_sc as plsc

N = 32768
E = 524288
D_IN = 4
D_H = 128
NB = 128
FIS = N // NB * D_H
NC = 2
NS = 16
FC = 16
NCHUNK = D_H // FC
ROWS = E // 128
NBLK = N // 16
HIGH = lax.Precision.HIGHEST

def _sc_mesh():
    return plsc.VectorSubcoreMesh(
        core_axis_name="c", subcore_axis_name="s",
        num_cores=NC, num_subcores=NS)


def _deg_body(dst_hbm, ones_hbm, zero1_hbm, out_hbm, idx_v, ones_v, acc_sh):
    c = lax.axis_index("c")
    s = lax.axis_index("s")
    pltpu.sync_copy(zero1_hbm, acc_sh.at[pl.ds(s * NBLK, NBLK)])
    pltpu.sync_copy(ones_hbm, ones_v)
    row0 = c * (ROWS // NC) + s * 128
    pltpu.sync_copy(dst_hbm.at[pl.ds(row0, 128)], idx_v)
    plsc.subcore_barrier()

    def body(j, carry):
        pltpu.sync_copy(ones_v, acc_sh.at[idx_v.at[j]], add=True)
        return carry

    lax.fori_loop(0, 128, body, 0)
    plsc.subcore_barrier()
    pltpu.sync_copy(acc_sh.at[pl.ds(s * NBLK, NBLK)],
                    out_hbm.at[c, pl.ds(s * NBLK, NBLK)])


def _sc_degree(dst2d, ones, zero1):
    return pl.kernel(
        _deg_body,
        out_type=jax.ShapeDtypeStruct((NC, N), jnp.float32),
        mesh=_sc_mesh(),
        scratch_types=[
            pltpu.VMEM((128, 128), jnp.int32),
            pltpu.VMEM((128,), jnp.float32),
            pltpu.VMEM_SHARED((N,), jnp.float32),
        ],
    )(dst2d, ones, zero1)


_NBUF = 16
_LOOK = 12


def _prop_body(g_hbm, src_hbm, dst_hbm, zero_hbm, out_hbm,
               src_v, dst_v, stage, acc_sh, gsem, ssem):
    c = lax.axis_index("c")
    s = lax.axis_index("s")
    nrow = ROWS // NS
    pltpu.sync_copy(src_hbm.at[pl.ds(s * nrow, nrow)], src_v)
    pltpu.sync_copy(dst_hbm.at[pl.ds(s * nrow, nrow)], dst_v)
    for k in range(NCHUNK // NC):
        chunk = c * (NCHUNK // NC) + k
        gk = g_hbm.at[chunk]
        for z in range(NBLK // 512):
            pltpu.sync_copy(zero_hbm, acc_sh.at[pl.ds(s * NBLK + z * 512, 512)])
        plsc.subcore_barrier()

        def issue_gather(j):
            pltpu.async_copy(gk.at[src_v.at[j]],
                             stage.at[lax.rem(j, _NBUF)], gsem)

        def wait_gather(j):
            pltpu.make_async_copy(gk.at[src_v.at[j]],
                                  stage.at[lax.rem(j, _NBUF)], gsem).wait()

        def issue_scatter(j):
            pltpu.async_copy(stage.at[lax.rem(j, _NBUF)],
                             acc_sh.at[dst_v.at[j]], ssem, add=True)

        def wait_one_scatter(j):
            pltpu.make_async_copy(stage.at[lax.rem(j, _NBUF)],
                                  acc_sh.at[dst_v.at[j]], ssem).wait()

        for j in range(_LOOK):
            issue_gather(j)

        def body(j, carry):
            @pl.when(j >= _NBUF - _LOOK)
            def _free():
                wait_one_scatter(j)

            @pl.when(j + _LOOK < nrow)
            def _ahead():
                issue_gather(j + _LOOK)

            wait_gather(j)
            issue_scatter(j)
            return carry

        lax.fori_loop(0, nrow, body, 0)
        for j in range(_NBUF - _LOOK):
            wait_one_scatter(j)
        plsc.subcore_barrier()
        pltpu.sync_copy(acc_sh.at[pl.ds(s * NBLK, NBLK)],
                        out_hbm.at[chunk, pl.ds(s * NBLK, NBLK)])
        if k + 1 < NCHUNK // NC:
            plsc.subcore_barrier()


def _sc_prop(gc, src2d, dst2d, zero2):
    return pl.kernel(
        _prop_body,
        out_type=jax.ShapeDtypeStruct((NCHUNK, N, FC), jnp.float32),
        mesh=_sc_mesh(),
        compiler_params=pltpu.CompilerParams(use_tc_tiling_on_sc=False),
        scratch_types=[
            pltpu.VMEM((ROWS // NS, 128), jnp.int32),
            pltpu.VMEM((ROWS // NS, 128), jnp.int32),
            pltpu.VMEM((_NBUF, 128, FC), jnp.float32),
            pltpu.VMEM_SHARED((N, FC), jnp.float32),
            pltpu.SemaphoreType.DMA,
            pltpu.SemaphoreType.DMA,
        ],
    )(gc, src2d, dst2d, zero2)


def _tc_pre(degT, x, W1):
    def body(deg_ref, x_ref, w1_ref, g1_ref, dinv_ref):
        deg = jnp.sum(deg_ref[...], axis=1, keepdims=True) + 1.0
        dinv = lax.rsqrt(deg)
        h = jnp.dot(x_ref[...], w1_ref[...],
                    preferred_element_type=jnp.float32, precision=HIGH)
        g1_ref[...] = h * dinv
        dinv_ref[...] = dinv

    return pl.pallas_call(
        body,
        grid=(16,),
        in_specs=[
            pl.BlockSpec((NBLK, NC), lambda i: (i, 0)),
            pl.BlockSpec((NBLK, D_IN), lambda i: (i, 0)),
            pl.BlockSpec((D_IN, D_H), lambda i: (0, 0)),
        ],
        out_specs=[
            pl.BlockSpec((NBLK, D_H), lambda i: (i, 0)),
            pl.BlockSpec((NBLK, 1), lambda i: (i, 0)),
        ],
        out_shape=[
            jax.ShapeDtypeStruct((N, D_H), jnp.float32),
            jax.ShapeDtypeStruct((N, 1), jnp.float32),
        ],
    )(degT, x, W1)


def _tc_mid(S1, g1, dinv, b1, W2):
    def body(s_ref, g_ref, dinv_ref, b1_ref, w2_ref, g2_ref):
        h1 = jnp.maximum(
            dinv_ref[...] * (s_ref[...] + g_ref[...]) + b1_ref[...], 0.0)
        g2_ref[...] = dinv_ref[...] * jnp.dot(
            h1, w2_ref[...], preferred_element_type=jnp.float32, precision=HIGH)

    return pl.pallas_call(
        body,
        grid=(16,),
        in_specs=[
            pl.BlockSpec((NBLK, D_H), lambda i: (i, 0)),
            pl.BlockSpec((NBLK, D_H), lambda i: (i, 0)),
            pl.BlockSpec((NBLK, 1), lambda i: (i, 0)),
            pl.BlockSpec((1, D_H), lambda i: (0, 0)),
            pl.BlockSpec((D_H, D_H), lambda i: (0, 0)),
        ],
        out_specs=pl.BlockSpec((NBLK, D_H), lambda i: (i, 0)),
        out_shape=jax.ShapeDtypeStruct((N, D_H), jnp.float32),
    )(S1, g1, dinv, b1, W2)


def _tc_h2(S2, g2, dinv, b2):
    def body(s_ref, g_ref, dinv_ref, b2_ref, h2_ref):
        h2_ref[...] = jnp.maximum(
            dinv_ref[...] * (s_ref[...] + g_ref[...]) + b2_ref[...], 0.0)

    return pl.pallas_call(
        body,
        grid=(16,),
        in_specs=[
            pl.BlockSpec((NBLK, D_H), lambda i: (i, 0)),
            pl.BlockSpec((NBLK, D_H), lambda i: (i, 0)),
            pl.BlockSpec((NBLK, 1), lambda i: (i, 0)),
            pl.BlockSpec((1, D_H), lambda i: (0, 0)),
        ],
        out_specs=pl.BlockSpec((NBLK, D_H), lambda i: (i, 0)),
        out_shape=jax.ShapeDtypeStruct((N, D_H), jnp.float32),
    )(S2, g2, dinv, b2)


def _tc_heads(flat, Wab, ba1, bv1, Wa2, ba2, Wv2, bv2):
    nsteps = FIS // NBLK

    def body(flat_ref, wab_ref, ba1_ref, bv1_ref, wa2_ref, ba2_ref,
             wv2_ref, bv2_ref, act_ref, val_ref, acc_ref):
        i = pl.program_id(0)

        @pl.when(i == 0)
        def _init():
            acc_ref[...] = jnp.zeros_like(acc_ref)

        acc_ref[...] += jnp.dot(flat_ref[...], wab_ref[...],
                                preferred_element_type=jnp.float32,
                                precision=HIGH)

        @pl.when(i == nsteps - 1)
        def _fin():
            acc = acc_ref[...]
            a = jnp.maximum(acc[:, :64] + ba1_ref[...], 0.0)
            logits = jnp.dot(a, wa2_ref[...],
                             preferred_element_type=jnp.float32,
                             precision=HIGH) + ba2_ref[...]
            m = jnp.max(logits, axis=1, keepdims=True)
            lse = m + jnp.log(jnp.sum(jnp.exp(logits - m), axis=1,
                                      keepdims=True))
            act_ref[...] = logits - lse
            v = jnp.maximum(acc[:, 64:] + bv1_ref[...], 0.0)
            val_ref[...] = jnp.tanh(
                jnp.dot(v, wv2_ref[...], preferred_element_type=jnp.float32,
                        precision=HIGH) + bv2_ref[...])

    return pl.pallas_call(
        body,
        grid=(nsteps,),
        in_specs=[
            pl.BlockSpec((NB, NBLK), lambda i: (0, i)),
            pl.BlockSpec((NBLK, 2 * 64), lambda i: (i, 0)),
            pl.BlockSpec((1, 64), lambda i: (0, 0)),
            pl.BlockSpec((1, 64), lambda i: (0, 0)),
            pl.BlockSpec((64, 256), lambda i: (0, 0)),
            pl.BlockSpec((1, 256), lambda i: (0, 0)),
            pl.BlockSpec((64, 1), lambda i: (0, 0)),
            pl.BlockSpec((1, 1), lambda i: (0, 0)),
        ],
        out_specs=[
            pl.BlockSpec((NB, 256), lambda i: (0, 0)),
            pl.BlockSpec((NB, 1), lambda i: (0, 0)),
        ],
        out_shape=[
            jax.ShapeDtypeStruct((NB, 256), jnp.float32),
            jax.ShapeDtypeStruct((NB, 1), jnp.float32),
        ],
        scratch_shapes=[pltpu.VMEM((NB, 2 * 64), jnp.float32)],
    )(flat, Wab, ba1, bv1, Wa2, ba2, Wv2, bv2)


def kernel(x, edge_index, W1, b1, W2, b2, Wa1, ba1, Wa2, ba2, Wv1, bv1,
           Wv2, bv2):
    src2d = edge_index[0].reshape(ROWS, 128)
    dst2d = edge_index[1].reshape(ROWS, 128)
    ones = jnp.ones((128,), jnp.float32)
    zero1 = jnp.zeros((NBLK,), jnp.float32)
    zero2 = jnp.zeros((512, FC), jnp.float32)

    deg2 = _sc_degree(dst2d, ones, zero1)
    g1, dinv = _tc_pre(deg2.T, x, W1)

    g1c = jnp.moveaxis(g1.reshape(N, NCHUNK, FC), 1, 0)
    S1c = _sc_prop(g1c, src2d, dst2d, zero2)
    S1 = jnp.moveaxis(S1c, 0, 1).reshape(N, D_H)

    g2 = _tc_mid(S1, g1, dinv, b1.reshape(1, D_H), W2)
    g2c = jnp.moveaxis(g2.reshape(N, NCHUNK, FC), 1, 0)
    S2c = _sc_prop(g2c, src2d, dst2d, zero2)
    S2 = jnp.moveaxis(S2c, 0, 1).reshape(N, D_H)

    h2 = _tc_h2(S2, g2, dinv, b2.reshape(1, D_H))
    flat = h2.reshape(NB, FIS)
    Wab = jnp.concatenate([Wa1, Wv1], axis=1)
    x_act, x_val = _tc_heads(flat, Wab, ba1.reshape(1, 64),
                             bv1.reshape(1, 64), Wa2, ba2.reshape(1, 256),
                             Wv2, bv2.reshape(1, 1))
    return (x_act, x_val)
